# baseline (speedup 1.0000x reference)
import functools
import jax
import jax.numpy as jnp
from jax import lax
from jax.experimental import pallas as pl
from jax.experimental.pallas import tpu as pltpu
from jax.experimental.pallas import tpu_sc as plsc

EMBED_DIM = 64
FFN_DIM = 128
NUM_EXPERTS = 4
NEG_INF = float("-inf")


def _logits_kernel(x_ref, wg_ref, o_ref, wgs):
    @pl.when(pl.program_id(0) == 0)
    def _prep():
        wgs[:] = jnp.transpose(wg_ref[:], (1, 0))
    o_ref[:] = jax.lax.dot_general(
        wgs[:], x_ref[:], (((1,), (0,)), ((), ())),
        preferred_element_type=jnp.float32)


def _make_sc_router(T):
    info = plsc.get_sparse_core_info()
    NC, NS, L = info.num_cores, info.num_subcores, info.num_lanes
    NW = NC * NS
    per_w = T // NW
    n_iter = per_w // L
    mesh = plsc.VectorSubcoreMesh(core_axis_name="c", subcore_axis_name="s")

    @functools.partial(
        pl.kernel, mesh=mesh,
        out_type=jax.ShapeDtypeStruct((4, T), jnp.float32),
        scratch_types=[
            pltpu.VMEM((4, per_w), jnp.float32),
            pltpu.VMEM((4, per_w), jnp.float32),
        ],
    )
    def router(lt_hbm, out_hbm, lv, wv):
        wid = lax.axis_index("s") * NC + lax.axis_index("c")
        base = wid * per_w
        for r in range(4):
            pltpu.sync_copy(lt_hbm.at[r, pl.ds(base, per_w)], lv.at[r])

        def body(i, _):
            s = pl.ds(i * L, L)
            l0 = lv[0, s]
            l1 = lv[1, s]
            l2 = lv[2, s]
            l3 = lv[3, s]
            m1 = jnp.maximum(jnp.maximum(l0, l1), jnp.maximum(l2, l3))
            i0 = jnp.full((L,), 0, jnp.int32)
            i1_ = jnp.full((L,), 1, jnp.int32)
            i2_ = jnp.full((L,), 2, jnp.int32)
            i3_ = jnp.full((L,), 3, jnp.int32)
            idx1 = jnp.where(l0 == m1, i0,
                             jnp.where(l1 == m1, i1_,
                                       jnp.where(l2 == m1, i2_, i3_)))
            ninf = jnp.full((L,), NEG_INF, jnp.float32)
            l0m = jnp.where(idx1 == 0, ninf, l0)
            l1m = jnp.where(idx1 == 1, ninf, l1)
            l2m = jnp.where(idx1 == 2, ninf, l2)
            l3m = jnp.where(idx1 == 3, ninf, l3)
            m2 = jnp.maximum(jnp.maximum(l0m, l1m), jnp.maximum(l2m, l3m))
            idx2 = jnp.where(l0m == m2, i0,
                             jnp.where(l1m == m2, i1_,
                                       jnp.where(l2m == m2, i2_, i3_)))
            p1 = 1.0 / (1.0 + jnp.exp(m2 - m1))
            p2 = 1.0 - p1
            zero = jnp.zeros((L,), jnp.float32)
            for e in range(4):
                wv[e, s] = jnp.where(idx1 == e, p1,
                                     jnp.where(idx2 == e, p2, zero))
            return ()

        lax.fori_loop(0, n_iter, body, ())
        for r in range(4):
            pltpu.sync_copy(wv.at[r], out_hbm.at[r, pl.ds(base, per_w)])

    return router


def _ffn_kernel(x_ref, wt_ref, w1_ref, b1_ref, w2t_ref, b2_ref, o_ref,
                w1s, w2s, b1s, b2s):
    D, F, E = EMBED_DIM, FFN_DIM, NUM_EXPERTS

    @pl.when(pl.program_id(0) == 0)
    def _prep():
        w2s[:] = jnp.zeros((E * D, E * F), jnp.float32)
        for e in range(E):
            w1s[e * F:(e + 1) * F, :] = jnp.transpose(w1_ref[e], (1, 0))
            w2s[e * D:(e + 1) * D, e * F:(e + 1) * F] = w2t_ref[e]
            b1s[e * F:(e + 1) * F, 0:1] = jnp.transpose(b1_ref[e:e + 1, :],
                                                        (1, 0))
            b2s[e * D:(e + 1) * D, 0:1] = jnp.transpose(b2_ref[e:e + 1, :],
                                                        (1, 0))

    xb = x_ref[:]
    wT = wt_ref[:]
    hT = jax.lax.dot_general(
        w1s[:], xb, (((1,), (0,)), ((), ())),
        preferred_element_type=jnp.float32) + b1s[:]
    hT = jnp.maximum(hT, 0.0)
    out_aT = jax.lax.dot_general(
        w2s[:], hT, (((1,), (0,)), ((), ())),
        preferred_element_type=jnp.float32)
    row_e = jax.lax.broadcasted_iota(jnp.int32, (E * D, E), 0) // D
    col_e = jax.lax.broadcasted_iota(jnp.int32, (E * D, E), 1)
    ex = jnp.where(row_e == col_e, 1.0, 0.0).astype(jnp.float32)
    wcolT = jax.lax.dot_general(
        ex, wT, (((1,), (0,)), ((), ())),
        preferred_element_type=jnp.float32)
    scaled = (out_aT + b2s[:]) * wcolT
    o_ref[:] = (scaled[0:D, :] + scaled[D:2 * D, :]
                + scaled[2 * D:3 * D, :] + scaled[3 * D:4 * D, :])


def kernel(x, Wg, W1, b1, W2, b2):
    x = x.reshape(-1, x.shape[-1])
    T, D = x.shape
    E, _, F = W1.shape
    xT = x.T
    W2t = W2.transpose(0, 2, 1)

    TB = 1024
    grid = (T // TB,)

    lT = pl.pallas_call(
        _logits_kernel,
        grid=grid,
        in_specs=[
            pl.BlockSpec((D, TB), lambda i: (0, i)),
            pl.BlockSpec((D, E), lambda i: (0, 0)),
        ],
        out_specs=pl.BlockSpec((E, TB), lambda i: (0, i)),
        out_shape=jax.ShapeDtypeStruct((E, T), jnp.float32),
        scratch_shapes=[pltpu.VMEM((E, D), jnp.float32)],
        compiler_params=pltpu.CompilerParams(
            dimension_semantics=("arbitrary",)),
    )(xT, Wg)

    wT = _make_sc_router(T)(lT)

    outT = pl.pallas_call(
        _ffn_kernel,
        grid=grid,
        in_specs=[
            pl.BlockSpec((D, TB), lambda i: (0, i)),
            pl.BlockSpec((E, TB), lambda i: (0, i)),
            pl.BlockSpec((E, D, F), lambda i: (0, 0, 0)),
            pl.BlockSpec((E, F), lambda i: (0, 0)),
            pl.BlockSpec((E, D, F), lambda i: (0, 0, 0)),
            pl.BlockSpec((E, D), lambda i: (0, 0)),
        ],
        out_specs=pl.BlockSpec((D, TB), lambda i: (0, i)),
        out_shape=jax.ShapeDtypeStruct((D, T), jnp.float32),
        scratch_shapes=[
            pltpu.VMEM((E * F, D), jnp.float32),
            pltpu.VMEM((E * D, E * F), jnp.float32),
            pltpu.VMEM((E * F, 1), jnp.float32),
            pltpu.VMEM((E * D, 1), jnp.float32),
        ],
        compiler_params=pltpu.CompilerParams(
            dimension_semantics=("arbitrary",)),
    )(xT, wT, W1, b1, W2t, b2)
    return outT.T


# speedup vs baseline: 2.4574x; 2.4574x over previous
import jax
import jax.numpy as jnp
from jax.experimental import pallas as pl
from jax.experimental.pallas import tpu as pltpu

EMBED_DIM = 64
FFN_DIM = 128
NUM_EXPERTS = 4


def _moe_kernel(x_ref, wg_ref, w1_ref, b1_ref, w2t_ref, b2_ref, o_ref,
                wgs, w1s, w2s, b1s, b2s):
    D, F, E = EMBED_DIM, FFN_DIM, NUM_EXPERTS

    @pl.when(pl.program_id(0) == 0)
    def _prep():
        wgs[:] = jnp.transpose(wg_ref[:], (1, 0))
        for e in range(E):
            w1s[e * F:(e + 1) * F, :] = jnp.transpose(w1_ref[e], (1, 0))
            w2s[e * D:(e + 1) * D, :] = w2t_ref[e]
            b1s[e * F:(e + 1) * F, 0:1] = jnp.transpose(b1_ref[e:e + 1, :],
                                                        (1, 0))
            b2s[e * D:(e + 1) * D, 0:1] = jnp.transpose(b2_ref[e:e + 1, :],
                                                        (1, 0))

    xb = x_ref[:]
    lT = jax.lax.dot_general(
        wgs[:], xb, (((1,), (0,)), ((), ())),
        preferred_element_type=jnp.float32)

    e_iota = jax.lax.broadcasted_iota(jnp.int32, lT.shape, 0)
    m1 = jnp.max(lT, axis=0, keepdims=True)
    idx1 = jnp.min(jnp.where(lT == m1, e_iota, E), axis=0, keepdims=True)
    masked = jnp.where(e_iota == idx1, -jnp.inf, lT)
    m2 = jnp.max(masked, axis=0, keepdims=True)
    idx2 = jnp.min(jnp.where(masked == m2, e_iota, E), axis=0, keepdims=True)
    p1 = 1.0 / (1.0 + jnp.exp(m2 - m1))
    p2 = 1.0 - p1
    wT = (jnp.where(e_iota == idx1, p1, 0.0)
          + jnp.where(e_iota == idx2, p2, 0.0))

    hT = jax.lax.dot_general(
        w1s[:], xb, (((1,), (0,)), ((), ())),
        preferred_element_type=jnp.float32) + b1s[:]
    hT = jnp.maximum(hT, 0.0)

    acc = None
    for e in range(E):
        out_e = jax.lax.dot_general(
            w2s[e * D:(e + 1) * D, :], hT[e * F:(e + 1) * F, :],
            (((1,), (0,)), ((), ())),
            preferred_element_type=jnp.float32)
        term = wT[e:e + 1, :] * (out_e + b2s[e * D:(e + 1) * D, :])
        acc = term if acc is None else acc + term
    o_ref[:] = acc


def kernel(x, Wg, W1, b1, W2, b2):
    x = x.reshape(-1, x.shape[-1])
    T, D = x.shape
    E, _, F = W1.shape
    xT = x.T
    W2t = W2.transpose(0, 2, 1)

    TB = 1024
    grid = (T // TB,)
    outT = pl.pallas_call(
        _moe_kernel,
        grid=grid,
        in_specs=[
            pl.BlockSpec((D, TB), lambda i: (0, i)),
            pl.BlockSpec((D, E), lambda i: (0, 0)),
            pl.BlockSpec((E, D, F), lambda i: (0, 0, 0)),
            pl.BlockSpec((E, F), lambda i: (0, 0)),
            pl.BlockSpec((E, D, F), lambda i: (0, 0, 0)),
            pl.BlockSpec((E, D), lambda i: (0, 0)),
        ],
        out_specs=pl.BlockSpec((D, TB), lambda i: (0, i)),
        out_shape=jax.ShapeDtypeStruct((D, T), jnp.float32),
        scratch_shapes=[
            pltpu.VMEM((E, D), jnp.float32),
            pltpu.VMEM((E * F, D), jnp.float32),
            pltpu.VMEM((E * D, F), jnp.float32),
            pltpu.VMEM((E * F, 1), jnp.float32),
            pltpu.VMEM((E * D, 1), jnp.float32),
        ],
        compiler_params=pltpu.CompilerParams(
            dimension_semantics=("arbitrary",)),
    )(xT, Wg, W1, b1, W2t, b2)
    return outT.T


# speedup vs baseline: 3.2637x; 1.3281x over previous
import jax
import jax.numpy as jnp
from jax.experimental import pallas as pl
from jax.experimental.pallas import tpu as pltpu

EMBED_DIM = 64
FFN_DIM = 128
NUM_EXPERTS = 4


def _moe_kernel(x_ref, wg_ref, w1_ref, b1_ref, w2t_ref, b2_ref, o_ref,
                wgs, w1s, w2s, b1s, b2s):
    D, F, E = EMBED_DIM, FFN_DIM, NUM_EXPERTS

    @pl.when(pl.program_id(0) == 0)
    def _prep():
        wgs[:] = jnp.transpose(wg_ref[:], (1, 0))
        for e in range(E):
            w1s[e * F:(e + 1) * F, :] = jnp.transpose(w1_ref[e], (1, 0))
            w2s[e * D:(e + 1) * D, :] = w2t_ref[e]
            b1s[e * F:(e + 1) * F, 0:1] = jnp.transpose(b1_ref[e:e + 1, :],
                                                        (1, 0))
            b2s[e * D:(e + 1) * D, 0:1] = jnp.transpose(b2_ref[e:e + 1, :],
                                                        (1, 0))

    HALF = x_ref.shape[1] // 2
    for h in range(2):
        sl = pl.ds(h * HALF, HALF)
        xb = x_ref[:, sl]
        lT = jax.lax.dot_general(
            wgs[:], xb, (((1,), (0,)), ((), ())),
            preferred_element_type=jnp.float32)

        e_iota = jax.lax.broadcasted_iota(jnp.int32, lT.shape, 0)
        m1 = jnp.max(lT, axis=0, keepdims=True)
        idx1 = jnp.min(jnp.where(lT == m1, e_iota, E), axis=0, keepdims=True)
        masked = jnp.where(e_iota == idx1, -jnp.inf, lT)
        m2 = jnp.max(masked, axis=0, keepdims=True)
        idx2 = jnp.min(jnp.where(masked == m2, e_iota, E),
                       axis=0, keepdims=True)
        p1 = 1.0 / (1.0 + jnp.exp(m2 - m1))
        p2 = 1.0 - p1
        wT = (jnp.where(e_iota == idx1, p1, 0.0)
              + jnp.where(e_iota == idx2, p2, 0.0))

        hT = jax.lax.dot_general(
            w1s[:], xb, (((1,), (0,)), ((), ())),
            preferred_element_type=jnp.float32) + b1s[:]
        hT = jnp.maximum(hT, 0.0)

        acc = None
        for e in range(E):
            out_e = jax.lax.dot_general(
                w2s[e * D:(e + 1) * D, :], hT[e * F:(e + 1) * F, :],
                (((1,), (0,)), ((), ())),
                preferred_element_type=jnp.float32)
            term = wT[e:e + 1, :] * (out_e + b2s[e * D:(e + 1) * D, :])
            acc = term if acc is None else acc + term
        o_ref[:, sl] = acc


def kernel(x, Wg, W1, b1, W2, b2):
    x = x.reshape(-1, x.shape[-1])
    T, D = x.shape
    E, _, F = W1.shape
    xT = x.T
    W2t = W2.transpose(0, 2, 1)

    TB = 2048
    grid = (T // TB,)
    outT = pl.pallas_call(
        _moe_kernel,
        grid=grid,
        in_specs=[
            pl.BlockSpec((D, TB), lambda i: (0, i)),
            pl.BlockSpec((D, E), lambda i: (0, 0)),
            pl.BlockSpec((E, D, F), lambda i: (0, 0, 0)),
            pl.BlockSpec((E, F), lambda i: (0, 0)),
            pl.BlockSpec((E, D, F), lambda i: (0, 0, 0)),
            pl.BlockSpec((E, D), lambda i: (0, 0)),
        ],
        out_specs=pl.BlockSpec((D, TB), lambda i: (0, i)),
        out_shape=jax.ShapeDtypeStruct((D, T), jnp.float32),
        scratch_shapes=[
            pltpu.VMEM((E, D), jnp.float32),
            pltpu.VMEM((E * F, D), jnp.float32),
            pltpu.VMEM((E * D, F), jnp.float32),
            pltpu.VMEM((E * F, 1), jnp.float32),
            pltpu.VMEM((E * D, 1), jnp.float32),
        ],
        compiler_params=pltpu.CompilerParams(
            dimension_semantics=("arbitrary",)),
    )(xT, Wg, W1, b1, W2t, b2)
    return outT.T


# speedup vs baseline: 3.5737x; 1.0950x over previous
import jax
import jax.numpy as jnp
from jax.experimental import pallas as pl
from jax.experimental.pallas import tpu as pltpu

EMBED_DIM = 64
FFN_DIM = 128
NUM_EXPERTS = 4


def _moe_kernel(x_ref, wg_ref, w1_ref, b1_ref, w2t_ref, b2_ref, o_ref,
                wgs, w1s, w2s, b1s, b2s):
    D, F, E = EMBED_DIM, FFN_DIM, NUM_EXPERTS

    @pl.when(pl.program_id(0) == 0)
    def _prep():
        wgs[:] = jnp.transpose(wg_ref[:], (1, 0))
        for e in range(E):
            w1s[e * F:(e + 1) * F, :] = jnp.transpose(w1_ref[e], (1, 0))
            w2s[e * D:(e + 1) * D, :] = w2t_ref[e]
            b1s[e * F:(e + 1) * F, 0:1] = jnp.transpose(b1_ref[e:e + 1, :],
                                                        (1, 0))
            b2s[e * D:(e + 1) * D, 0:1] = jnp.transpose(b2_ref[e:e + 1, :],
                                                        (1, 0))

    NSUB = 4
    HALF = x_ref.shape[1] // NSUB
    for h in range(NSUB):
        sl = pl.ds(h * HALF, HALF)
        xb = x_ref[:, sl]
        lT = jax.lax.dot_general(
            wgs[:], xb, (((1,), (0,)), ((), ())),
            preferred_element_type=jnp.float32)

        e_iota = jax.lax.broadcasted_iota(jnp.int32, lT.shape, 0)
        m1 = jnp.max(lT, axis=0, keepdims=True)
        idx1 = jnp.min(jnp.where(lT == m1, e_iota, E), axis=0, keepdims=True)
        masked = jnp.where(e_iota == idx1, -jnp.inf, lT)
        m2 = jnp.max(masked, axis=0, keepdims=True)
        idx2 = jnp.min(jnp.where(masked == m2, e_iota, E),
                       axis=0, keepdims=True)
        p1 = 1.0 / (1.0 + jnp.exp(m2 - m1))
        p2 = 1.0 - p1
        wT = (jnp.where(e_iota == idx1, p1, 0.0)
              + jnp.where(e_iota == idx2, p2, 0.0))

        hT = jax.lax.dot_general(
            w1s[:], xb, (((1,), (0,)), ((), ())),
            preferred_element_type=jnp.float32) + b1s[:]
        hT = jnp.maximum(hT, 0.0)

        acc = None
        for e in range(E):
            out_e = jax.lax.dot_general(
                w2s[e * D:(e + 1) * D, :], hT[e * F:(e + 1) * F, :],
                (((1,), (0,)), ((), ())),
                preferred_element_type=jnp.float32)
            term = wT[e:e + 1, :] * (out_e + b2s[e * D:(e + 1) * D, :])
            acc = term if acc is None else acc + term
        o_ref[:, sl] = acc


def kernel(x, Wg, W1, b1, W2, b2):
    x = x.reshape(-1, x.shape[-1])
    T, D = x.shape
    E, _, F = W1.shape
    xT = x.T
    W2t = W2.transpose(0, 2, 1)

    TB = 4096
    grid = (T // TB,)
    outT = pl.pallas_call(
        _moe_kernel,
        grid=grid,
        in_specs=[
            pl.BlockSpec((D, TB), lambda i: (0, i)),
            pl.BlockSpec((D, E), lambda i: (0, 0)),
            pl.BlockSpec((E, D, F), lambda i: (0, 0, 0)),
            pl.BlockSpec((E, F), lambda i: (0, 0)),
            pl.BlockSpec((E, D, F), lambda i: (0, 0, 0)),
            pl.BlockSpec((E, D), lambda i: (0, 0)),
        ],
        out_specs=pl.BlockSpec((D, TB), lambda i: (0, i)),
        out_shape=jax.ShapeDtypeStruct((D, T), jnp.float32),
        scratch_shapes=[
            pltpu.VMEM((E, D), jnp.float32),
            pltpu.VMEM((E * F, D), jnp.float32),
            pltpu.VMEM((E * D, F), jnp.float32),
            pltpu.VMEM((E * F, 1), jnp.float32),
            pltpu.VMEM((E * D, 1), jnp.float32),
        ],
        compiler_params=pltpu.CompilerParams(
            dimension_semantics=("arbitrary",)),
    )(xT, Wg, W1, b1, W2t, b2)
    return outT.T


# speedup vs baseline: 3.5879x; 1.0040x over previous
import jax
import jax.numpy as jnp
from jax.experimental import pallas as pl
from jax.experimental.pallas import tpu as pltpu

EMBED_DIM = 64
FFN_DIM = 128
NUM_EXPERTS = 4


def _moe_kernel(x_ref, wg_ref, w1_ref, b1_ref, w2t_ref, b2_ref, o_ref,
                wgs, w1s, w2s, b1s, b2s):
    D, F, E = EMBED_DIM, FFN_DIM, NUM_EXPERTS

    @pl.when(pl.program_id(0) == 0)
    def _prep():
        wgs[:] = jnp.transpose(wg_ref[:], (1, 0))
        for e in range(E):
            w1s[e * F:(e + 1) * F, :] = jnp.transpose(w1_ref[e], (1, 0))
            w2s[e * D:(e + 1) * D, :] = w2t_ref[e]
            b1s[e * F:(e + 1) * F, 0:1] = jnp.transpose(b1_ref[e:e + 1, :],
                                                        (1, 0))
            b2s[e * D:(e + 1) * D, 0:1] = jnp.transpose(b2_ref[e:e + 1, :],
                                                        (1, 0))

    NSUB = 8
    HALF = x_ref.shape[1] // NSUB
    for h in range(NSUB):
        sl = pl.ds(h * HALF, HALF)
        xb = x_ref[:, sl]
        lT = jax.lax.dot_general(
            wgs[:], xb, (((1,), (0,)), ((), ())),
            preferred_element_type=jnp.float32)

        e_iota = jax.lax.broadcasted_iota(jnp.int32, lT.shape, 0)
        m1 = jnp.max(lT, axis=0, keepdims=True)
        idx1 = jnp.min(jnp.where(lT == m1, e_iota, E), axis=0, keepdims=True)
        masked = jnp.where(e_iota == idx1, -jnp.inf, lT)
        m2 = jnp.max(masked, axis=0, keepdims=True)
        idx2 = jnp.min(jnp.where(masked == m2, e_iota, E),
                       axis=0, keepdims=True)
        p1 = 1.0 / (1.0 + jnp.exp(m2 - m1))
        p2 = 1.0 - p1
        wT = (jnp.where(e_iota == idx1, p1, 0.0)
              + jnp.where(e_iota == idx2, p2, 0.0))

        hT = jax.lax.dot_general(
            w1s[:], xb, (((1,), (0,)), ((), ())),
            preferred_element_type=jnp.float32) + b1s[:]
        hT = jnp.maximum(hT, 0.0)

        acc = None
        for e in range(E):
            out_e = jax.lax.dot_general(
                w2s[e * D:(e + 1) * D, :], hT[e * F:(e + 1) * F, :],
                (((1,), (0,)), ((), ())),
                preferred_element_type=jnp.float32)
            term = wT[e:e + 1, :] * (out_e + b2s[e * D:(e + 1) * D, :])
            acc = term if acc is None else acc + term
        o_ref[:, sl] = acc


def kernel(x, Wg, W1, b1, W2, b2):
    x = x.reshape(-1, x.shape[-1])
    T, D = x.shape
    E, _, F = W1.shape
    xT = x.T
    W2t = W2.transpose(0, 2, 1)

    TB = 8192
    grid = (T // TB,)
    outT = pl.pallas_call(
        _moe_kernel,
        grid=grid,
        in_specs=[
            pl.BlockSpec((D, TB), lambda i: (0, i)),
            pl.BlockSpec((D, E), lambda i: (0, 0)),
            pl.BlockSpec((E, D, F), lambda i: (0, 0, 0)),
            pl.BlockSpec((E, F), lambda i: (0, 0)),
            pl.BlockSpec((E, D, F), lambda i: (0, 0, 0)),
            pl.BlockSpec((E, D), lambda i: (0, 0)),
        ],
        out_specs=pl.BlockSpec((D, TB), lambda i: (0, i)),
        out_shape=jax.ShapeDtypeStruct((D, T), jnp.float32),
        scratch_shapes=[
            pltpu.VMEM((E, D), jnp.float32),
            pltpu.VMEM((E * F, D), jnp.float32),
            pltpu.VMEM((E * D, F), jnp.float32),
            pltpu.VMEM((E * F, 1), jnp.float32),
            pltpu.VMEM((E * D, 1), jnp.float32),
        ],
        compiler_params=pltpu.CompilerParams(
            dimension_semantics=("arbitrary",)),
    )(xT, Wg, W1, b1, W2t, b2)
    return outT.T
